# baseline (speedup 1.0000x reference)
import functools

import numpy as np
import jax
import jax.numpy as jnp
from jax import lax
from jax.experimental import pallas as pl
from jax.experimental.pallas import tpu as pltpu
from jax.experimental.pallas import tpu_sc as plsc

VOCAB = 100000
SEQ_LEN = 2048
D_MODEL = 768
N_BASE = 10000
BATCH = 4

NUM_CORES = 2
NUM_SUBCORES = 16
LANES = 16
NW = NUM_CORES * NUM_SUBCORES
TOTAL = BATCH * SEQ_LEN
PER_W = TOTAL // NW
CHUNK = 64
NCHUNK = PER_W // CHUNK
NVREG = D_MODEL // LANES


def _positional_encoding():
    depth = D_MODEL // 2
    positions = np.arange(SEQ_LEN)[:, np.newaxis]
    depths = np.arange(depth)[np.newaxis, :] / depth
    angle_rads = positions * (1.0 / N_BASE ** depths)
    enc = np.zeros((SEQ_LEN, D_MODEL), dtype=np.float32)
    enc[:, 0::2] = np.sin(angle_rads)
    enc[:, 1::2] = np.cos(angle_rads)
    return enc


_POS_ENC = _positional_encoding()

_mesh = plsc.VectorSubcoreMesh(core_axis_name="c", subcore_axis_name="s")


@functools.partial(
    pl.kernel,
    out_type=jax.ShapeDtypeStruct((TOTAL, D_MODEL), jnp.float32),
    mesh=_mesh,
    scratch_types=[
        pltpu.VMEM((CHUNK,), jnp.int32),
        pltpu.VMEM((CHUNK, D_MODEL), jnp.float32),
        pltpu.VMEM((CHUNK, D_MODEL), jnp.float32),
        pltpu.SemaphoreType.DMA,
    ],
)
def _emb_lookup(idx_hbm, table_hbm, pos_hbm, out_hbm, idx_v, pos_v, rows_v, sem):
    wid = lax.axis_index("s") * NUM_CORES + lax.axis_index("c")
    base = wid * PER_W
    s0 = lax.rem(base, SEQ_LEN)
    for c in range(NCHUNK):
        offs = base + c * CHUNK
        pltpu.sync_copy(idx_hbm.at[pl.ds(offs, CHUNK)], idx_v)
        gather = pltpu.async_copy(table_hbm.at[idx_v], rows_v, sem)
        pltpu.sync_copy(pos_hbm.at[pl.ds(s0 + c * CHUNK, CHUNK), :], pos_v)
        gather.wait()

        def _row_add(r, carry):
            for j in range(NVREG):
                sl = pl.ds(j * LANES, LANES)
                pos_v[r, sl] = pos_v[r, sl] + rows_v[r, sl]
            return carry

        lax.fori_loop(0, CHUNK, _row_add, 0)
        pltpu.sync_copy(pos_v, out_hbm.at[pl.ds(offs, CHUNK), :])


def kernel(x, table):
    xflat = x.reshape(TOTAL).astype(jnp.int32)
    pos = jnp.asarray(_POS_ENC)
    out = _emb_lookup(xflat, table, pos)
    return out.reshape(BATCH, SEQ_LEN, D_MODEL)


# speedup vs baseline: 1.0961x; 1.0961x over previous
import functools

import numpy as np
import jax
import jax.numpy as jnp
from jax import lax
from jax.experimental import pallas as pl
from jax.experimental.pallas import tpu as pltpu
from jax.experimental.pallas import tpu_sc as plsc

VOCAB = 100000
SEQ_LEN = 2048
D_MODEL = 768
N_BASE = 10000
BATCH = 4

NUM_CORES = 2
NUM_SUBCORES = 16
LANES = 16
NW = NUM_CORES * NUM_SUBCORES
TOTAL = BATCH * SEQ_LEN
S_PER_W = SEQ_LEN // NW
PER_W = S_PER_W * BATCH
SUB = 32
NSUB = PER_W // SUB
NBUF = 3
NVREG = D_MODEL // LANES


def _positional_encoding():
    depth = D_MODEL // 2
    positions = np.arange(SEQ_LEN)[:, np.newaxis]
    depths = np.arange(depth)[np.newaxis, :] / depth
    angle_rads = positions * (1.0 / N_BASE ** depths)
    enc = np.zeros((SEQ_LEN, D_MODEL), dtype=np.float32)
    enc[:, 0::2] = np.sin(angle_rads)
    enc[:, 1::2] = np.cos(angle_rads)
    return enc


_POS_ENC = _positional_encoding()

_mesh = plsc.VectorSubcoreMesh(core_axis_name="c", subcore_axis_name="s")


@functools.partial(
    pl.kernel,
    out_type=jax.ShapeDtypeStruct((TOTAL, D_MODEL), jnp.float32),
    mesh=_mesh,
    scratch_types=[
        pltpu.VMEM((PER_W,), jnp.int32),
        pltpu.VMEM((S_PER_W, D_MODEL), jnp.float32),
        [pltpu.VMEM((SUB, D_MODEL), jnp.float32) for _ in range(NBUF)],
        pltpu.SemaphoreType.DMA((NBUF,)),
        pltpu.SemaphoreType.DMA((NBUF,)),
    ],
)
def _emb_lookup(idx_hbm, table_hbm, pos_hbm, out_hbm,
                idx_v, pos_v, rows_v, sem_g, sem_s):
    wid = lax.axis_index("s") * NUM_CORES + lax.axis_index("c")
    s0 = wid * S_PER_W

    for b in range(BATCH):
        pltpu.sync_copy(idx_hbm.at[b, pl.ds(s0, S_PER_W)],
                        idx_v.at[pl.ds(b * S_PER_W, S_PER_W)])

    def gather(i):
        b, h = divmod(i, NSUB // BATCH)
        src = table_hbm.at[idx_v.at[pl.ds(b * S_PER_W + h * SUB, SUB)]]
        return pltpu.async_copy(src, rows_v[i % NBUF], sem_g.at[i % NBUF])

    gathers = {i: gather(i) for i in range(NBUF)}
    pltpu.sync_copy(pos_hbm.at[pl.ds(s0, S_PER_W), :], pos_v)

    stores = {}
    for i in range(NSUB):
        p = i % NBUF
        b, h = divmod(i, NSUB // BATCH)
        if 0 < i and i + 2 < NSUB:
            stores.pop(i - 1).wait()
            gathers[i + 2] = gather(i + 2)
        gathers.pop(i).wait()

        def _row_add(r, carry, h=h, p=p):
            for j in range(NVREG):
                sl = pl.ds(j * LANES, LANES)
                plsc.addupdate(rows_v[p].at[r, sl], pos_v[h * SUB + r, sl])
            return carry

        lax.fori_loop(0, SUB, _row_add, 0)
        dst = out_hbm.at[pl.ds(b * SEQ_LEN + s0 + h * SUB, SUB), :]
        stores[i] = pltpu.async_copy(rows_v[p], dst, sem_s.at[p])

    for d in stores.values():
        d.wait()


def kernel(x, table):
    xi = x.reshape(BATCH, SEQ_LEN).astype(jnp.int32)
    pos = jnp.asarray(_POS_ENC)
    out = _emb_lookup(xi, table, pos)
    return out.reshape(BATCH, SEQ_LEN, D_MODEL)


# speedup vs baseline: 1.1173x; 1.0194x over previous
import functools

import numpy as np
import jax
import jax.numpy as jnp
from jax import lax
from jax.experimental import pallas as pl
from jax.experimental.pallas import tpu as pltpu
from jax.experimental.pallas import tpu_sc as plsc

VOCAB = 100000
SEQ_LEN = 2048
D_MODEL = 768
N_BASE = 10000
BATCH = 4

NUM_CORES = 2
NUM_SUBCORES = 16
LANES = 16
NW = NUM_CORES * NUM_SUBCORES
TOTAL = BATCH * SEQ_LEN
S_PER_W = SEQ_LEN // NW
PER_W = S_PER_W * BATCH
SUB = 16
NSUB = PER_W // SUB
SUB_PER_B = S_PER_W // SUB
NBUF = 6
LOOKAHEAD = 2
NVREG = D_MODEL // LANES


def _positional_encoding():
    depth = D_MODEL // 2
    positions = np.arange(SEQ_LEN)[:, np.newaxis]
    depths = np.arange(depth)[np.newaxis, :] / depth
    angle_rads = positions * (1.0 / N_BASE ** depths)
    enc = np.zeros((SEQ_LEN, D_MODEL), dtype=np.float32)
    enc[:, 0::2] = np.sin(angle_rads)
    enc[:, 1::2] = np.cos(angle_rads)
    return enc


_POS_ENC = _positional_encoding()

_mesh = plsc.VectorSubcoreMesh(core_axis_name="c", subcore_axis_name="s")


@functools.partial(
    pl.kernel,
    out_type=jax.ShapeDtypeStruct((TOTAL, D_MODEL), jnp.float32),
    mesh=_mesh,
    scratch_types=[
        pltpu.VMEM((PER_W,), jnp.int32),
        pltpu.VMEM((S_PER_W, D_MODEL), jnp.float32),
        [pltpu.VMEM((SUB, D_MODEL), jnp.float32) for _ in range(NBUF)],
        pltpu.SemaphoreType.DMA((NBUF,)),
        pltpu.SemaphoreType.DMA((NBUF,)),
    ],
)
def _emb_lookup(idx_hbm, table_hbm, pos_hbm, out_hbm,
                idx_v, pos_v, rows_v, sem_g, sem_s):
    wid = lax.axis_index("s") * NUM_CORES + lax.axis_index("c")
    s0 = wid * S_PER_W

    for b in range(BATCH):
        pltpu.sync_copy(idx_hbm.at[b, pl.ds(s0, S_PER_W)],
                        idx_v.at[pl.ds(b * S_PER_W, S_PER_W)])

    def gather(i):
        b, h = divmod(i, SUB_PER_B)
        src = table_hbm.at[idx_v.at[pl.ds(b * S_PER_W + h * SUB, SUB)]]
        return pltpu.async_copy(src, rows_v[i % NBUF], sem_g.at[i % NBUF])

    gathers = {i: gather(i) for i in range(LOOKAHEAD + 1)}
    pltpu.sync_copy(pos_hbm.at[pl.ds(s0, S_PER_W), :], pos_v)

    stores = {}
    for i in range(NSUB):
        p = i % NBUF
        b, h = divmod(i, SUB_PER_B)
        nxt = i + LOOKAHEAD + 1
        if nxt < NSUB:
            if nxt - NBUF in stores:
                stores.pop(nxt - NBUF).wait()
            gathers[nxt] = gather(nxt)
        gathers.pop(i).wait()

        def _row_add(r, carry, h=h, p=p):
            for j in range(NVREG):
                sl = pl.ds(j * LANES, LANES)
                plsc.addupdate(rows_v[p].at[r, sl], pos_v[h * SUB + r, sl])
            return carry

        lax.fori_loop(0, SUB, _row_add, 0)
        dst = out_hbm.at[pl.ds(b * SEQ_LEN + s0 + h * SUB, SUB), :]
        stores[i] = pltpu.async_copy(rows_v[p], dst, sem_s.at[p])

    for d in stores.values():
        d.wait()


def kernel(x, table):
    xi = x.reshape(BATCH, SEQ_LEN).astype(jnp.int32)
    pos = jnp.asarray(_POS_ENC)
    out = _emb_lookup(xi, table, pos)
    return out.reshape(BATCH, SEQ_LEN, D_MODEL)


# speedup vs baseline: 1.3724x; 1.2283x over previous
import functools

import numpy as np
import jax
import jax.numpy as jnp
from jax import lax
from jax.experimental import pallas as pl
from jax.experimental.pallas import tpu as pltpu
from jax.experimental.pallas import tpu_sc as plsc

VOCAB = 100000
SEQ_LEN = 2048
D_MODEL = 768
N_BASE = 10000
BATCH = 4

NUM_CORES = 2
NUM_SUBCORES = 16
LANES = 16
NW = NUM_CORES * NUM_SUBCORES
TOTAL = BATCH * SEQ_LEN
S_PER_W = SEQ_LEN // NW
PER_W = S_PER_W * BATCH
SUB = 16
NGRP = S_PER_W // SUB
NVREG = D_MODEL // LANES


def _positional_encoding():
    depth = D_MODEL // 2
    positions = np.arange(SEQ_LEN)[:, np.newaxis]
    depths = np.arange(depth)[np.newaxis, :] / depth
    angle_rads = positions * (1.0 / N_BASE ** depths)
    enc = np.zeros((SEQ_LEN, D_MODEL), dtype=np.float32)
    enc[:, 0::2] = np.sin(angle_rads)
    enc[:, 1::2] = np.cos(angle_rads)
    return enc


_POS_ENC = _positional_encoding()

_mesh = plsc.VectorSubcoreMesh(core_axis_name="c", subcore_axis_name="s")


@functools.partial(
    pl.kernel,
    out_type=jax.ShapeDtypeStruct((TOTAL, D_MODEL), jnp.float32),
    mesh=_mesh,
    scratch_types=[
        pltpu.VMEM((PER_W,), jnp.int32),
        [pltpu.VMEM((SUB, D_MODEL), jnp.float32) for _ in range(2)],
        [[pltpu.VMEM((SUB, D_MODEL), jnp.float32) for _ in range(2)]
         for _ in range(BATCH)],
        pltpu.SemaphoreType.DMA((2,)),
        pltpu.SemaphoreType.DMA((BATCH, 2)),
        pltpu.SemaphoreType.DMA((BATCH, 2)),
    ],
)
def _emb_lookup(idx_hbm, table_hbm, pos_hbm, out_hbm,
                idx_v, pos_v, rows_v, sem_p, sem_g, sem_s):
    wid = lax.axis_index("s") * NUM_CORES + lax.axis_index("c")
    s0 = wid * S_PER_W

    for b in range(BATCH):
        pltpu.sync_copy(idx_hbm.at[b, pl.ds(s0, S_PER_W)],
                        idx_v.at[pl.ds(b * S_PER_W, S_PER_W)])

    def gather(g, b):
        src = table_hbm.at[idx_v.at[pl.ds(b * S_PER_W + g * SUB, SUB)]]
        return pltpu.async_copy(src, rows_v[b][g % 2], sem_g.at[b, g % 2])

    def pos_load(g):
        return pltpu.async_copy(pos_hbm.at[pl.ds(s0 + g * SUB, SUB), :],
                                pos_v[g % 2], sem_p.at[g % 2])

    pending = {g: [pos_load(g)] + [gather(g, b) for b in range(BATCH)]
               for g in range(2)}
    stores = {}
    for g in range(NGRP):
        q = g % 2
        if g >= 1 and g + 1 < NGRP:
            for d in stores.pop(g - 1):
                d.wait()
            pending[g + 1] = [pos_load(g + 1)] + \
                [gather(g + 1, b) for b in range(BATCH)]
        for d in pending.pop(g):
            d.wait()

        def _row_add(r, carry, q=q):
            for j in range(NVREG):
                sl = pl.ds(j * LANES, LANES)
                v = pos_v[q][r, sl]
                for b in range(BATCH):
                    plsc.addupdate(rows_v[b][q].at[r, sl], v)
            return carry

        lax.fori_loop(0, SUB, _row_add, 0)
        stores[g] = [
            pltpu.async_copy(
                rows_v[b][q],
                out_hbm.at[pl.ds(b * SEQ_LEN + s0 + g * SUB, SUB), :],
                sem_s.at[b, q])
            for b in range(BATCH)]

    for ds in stores.values():
        for d in ds:
            d.wait()


def kernel(x, table):
    xi = x.reshape(BATCH, SEQ_LEN).astype(jnp.int32)
    pos = jnp.asarray(_POS_ENC)
    out = _emb_lookup(xi, table, pos)
    return out.reshape(BATCH, SEQ_LEN, D_MODEL)


# speedup vs baseline: 1.4389x; 1.0484x over previous
import functools

import numpy as np
import jax
import jax.numpy as jnp
from jax import lax
from jax.experimental import pallas as pl
from jax.experimental.pallas import tpu as pltpu
from jax.experimental.pallas import tpu_sc as plsc

VOCAB = 100000
SEQ_LEN = 2048
D_MODEL = 768
N_BASE = 10000
BATCH = 4

NUM_CORES = 2
NUM_SUBCORES = 16
LANES = 16
NW = NUM_CORES * NUM_SUBCORES
TOTAL = BATCH * SEQ_LEN
S_PER_W = SEQ_LEN // NW
PER_W = S_PER_W * BATCH
SUB = 16
NGRP = S_PER_W // SUB
NVREG = D_MODEL // LANES


def _positional_encoding():
    depth = D_MODEL // 2
    positions = np.arange(SEQ_LEN)[:, np.newaxis]
    depths = np.arange(depth)[np.newaxis, :] / depth
    angle_rads = positions * (1.0 / N_BASE ** depths)
    enc = np.zeros((SEQ_LEN, D_MODEL), dtype=np.float32)
    enc[:, 0::2] = np.sin(angle_rads)
    enc[:, 1::2] = np.cos(angle_rads)
    return enc


_POS_ENC = _positional_encoding()

_mesh = plsc.VectorSubcoreMesh(core_axis_name="c", subcore_axis_name="s")


@functools.partial(
    pl.kernel,
    out_type=jax.ShapeDtypeStruct((TOTAL, D_MODEL), jnp.float32),
    mesh=_mesh,
    scratch_types=[
        pltpu.VMEM((BATCH, S_PER_W), jnp.int32),
        [pltpu.VMEM((SUB, D_MODEL), jnp.float32) for _ in range(2)],
        [[pltpu.VMEM((SUB, D_MODEL), jnp.float32) for _ in range(2)]
         for _ in range(BATCH)],
        pltpu.SemaphoreType.DMA((2,)),
        pltpu.SemaphoreType.DMA((BATCH, 2)),
        pltpu.SemaphoreType.DMA((BATCH, 2)),
        pltpu.SemaphoreType.DMA,
    ],
)
def _emb_lookup(idx_hbm, table_hbm, pos_hbm, out_hbm,
                idx_v, pos_v, rows_v, sem_p, sem_g, sem_s, sem_i):
    wid = lax.axis_index("s") * NUM_CORES + lax.axis_index("c")
    s0 = wid * S_PER_W

    idx_cps = [
        pltpu.async_copy(idx_hbm.at[b, pl.ds(s0, S_PER_W)], idx_v.at[b], sem_i)
        for b in range(BATCH)]

    def gather(g, b):
        src = table_hbm.at[idx_v.at[b, pl.ds(g * SUB, SUB)]]
        return pltpu.async_copy(src, rows_v[b][g % 2], sem_g.at[b, g % 2])

    def pos_load(g):
        return pltpu.async_copy(pos_hbm.at[pl.ds(s0 + g * SUB, SUB), :],
                                pos_v[g % 2], sem_p.at[g % 2])

    pos_pending = {g: pos_load(g) for g in range(2)}
    for cp in idx_cps:
        cp.wait()
    pending = {g: [gather(g, b) for b in range(BATCH)] for g in range(2)}
    stores = {}
    for g in range(NGRP):
        q = g % 2
        if g >= 1 and g + 1 < NGRP:
            pos_pending[g + 1] = pos_load(g + 1)
            for d in stores.pop(g - 1):
                d.wait()
            pending[g + 1] = [gather(g + 1, b) for b in range(BATCH)]
        pos_pending.pop(g).wait()
        for d in pending.pop(g):
            d.wait()

        def _row_add(r, carry, q=q):
            for j in range(NVREG):
                sl = pl.ds(j * LANES, LANES)
                v = pos_v[q][r, sl]
                for b in range(BATCH):
                    plsc.addupdate(rows_v[b][q].at[r, sl], v)
            return carry

        lax.fori_loop(0, SUB, _row_add, 0)
        stores[g] = [
            pltpu.async_copy(
                rows_v[b][q],
                out_hbm.at[pl.ds(b * SEQ_LEN + s0 + g * SUB, SUB), :],
                sem_s.at[b, q])
            for b in range(BATCH)]

    for ds in stores.values():
        for d in ds:
            d.wait()


def kernel(x, table):
    xi = x.reshape(BATCH, SEQ_LEN).astype(jnp.int32)
    pos = jnp.asarray(_POS_ENC)
    out = _emb_lookup(xi, table, pos)
    return out.reshape(BATCH, SEQ_LEN, D_MODEL)
